# Initial kernel scaffold; baseline (speedup 1.0000x reference)
import jax
import jax.numpy as jnp
from jax.experimental import pallas as pl


def kernel(x, edge_index, edge_weight, W1, W2):
    raise NotImplementedError("write your pallas kernel here")


# speedup vs baseline: 10.3190x; 10.3190x over previous
import functools

import jax
import jax.numpy as jnp
from jax import lax
from jax.experimental import pallas as pl
from jax.experimental.pallas import tpu as pltpu
import jax.experimental.pallas.tpu_sc as plsc

N_NODES = 10000
N_EDGES = 320000
DIM = 128

NC = 2
NS = 16
L = 16

CH = 80
CHUNKS_PER_TILE = N_EDGES // (NC * NS * CH)
ROWS_PER_TILE = N_NODES // NS
ZROWS = 5


def _spmm_body(x_hbm, src_hbm, dst_hbm, w_hbm, out_hbm,
               idx_s, db0, db1, wb0, wb1, rows0, rows1, zbuf, acc,
               sg0, sg1, sw0, sw1, sd0, sd1):
    c = lax.axis_index("c")
    s = lax.axis_index("s")
    wid = c * NS + s

    pltpu.sync_copy(src_hbm.at[wid], idx_s)
    cp_g0 = pltpu.async_copy(x_hbm.at[idx_s.at[0]], rows0, sg0)
    cp_g1 = pltpu.async_copy(x_hbm.at[idx_s.at[1]], rows1, sg1)
    cp_w0 = pltpu.async_copy(w_hbm.at[wid, 0], wb0, sw0)
    cp_w1 = pltpu.async_copy(w_hbm.at[wid, 1], wb1, sw1)
    cp_d0 = pltpu.async_copy(dst_hbm.at[wid, 0], db0, sd0)
    cp_d1 = pltpu.async_copy(dst_hbm.at[wid, 1], db1, sd1)

    zero16 = jnp.zeros((L,), jnp.float32)
    for i in range(ZROWS):
        for k in range(DIM // L):
            zbuf[i, pl.ds(k * L, L)] = zero16
    base_r = s * ROWS_PER_TILE

    def zloop(j, carry):
        pltpu.sync_copy(zbuf, acc.at[pl.ds(base_r + j * ZROWS, ZROWS)])
        return carry

    lax.fori_loop(0, ROWS_PER_TILE // ZROWS, zloop, 0)
    plsc.subcore_barrier()

    def scale(rows_b, wb):
        def grp_body(eg, carry):
            wv = wb[pl.ds(eg * L, L)]
            for j in range(L):
                we = wv[j]
                e = eg * L + j
                for k in range(DIM // L):
                    sl = pl.ds(k * L, L)
                    rows_b[e, sl] = rows_b[e, sl] * we
            return carry

        lax.fori_loop(0, CH // L, grp_body, 0)

    def pipe_body(g2, carry):
        g = 2 * g2
        cp_g0.wait()
        cp_w0.wait()
        cp_d0.wait()
        scale(rows0, wb0)
        pltpu.sync_copy(rows0, acc.at[db0], add=True)
        pltpu.async_copy(x_hbm.at[idx_s.at[g + 2]], rows0, sg0)
        pltpu.async_copy(w_hbm.at[wid, g + 2], wb0, sw0)
        pltpu.async_copy(dst_hbm.at[wid, g + 2], db0, sd0)

        cp_g1.wait()
        cp_w1.wait()
        cp_d1.wait()
        scale(rows1, wb1)
        pltpu.sync_copy(rows1, acc.at[db1], add=True)

        @pl.when(g2 < (CHUNKS_PER_TILE - 1) // 2 - 1)
        def _():
            pltpu.async_copy(x_hbm.at[idx_s.at[g + 3]], rows1, sg1)
            pltpu.async_copy(w_hbm.at[wid, g + 3], wb1, sw1)
            pltpu.async_copy(dst_hbm.at[wid, g + 3], db1, sd1)

        return carry

    lax.fori_loop(0, (CHUNKS_PER_TILE - 1) // 2, pipe_body, 0)

    cp_g0.wait()
    cp_w0.wait()
    cp_d0.wait()
    scale(rows0, wb0)
    pltpu.sync_copy(rows0, acc.at[db0], add=True)

    plsc.subcore_barrier()

    pltpu.sync_copy(acc.at[pl.ds(base_r, ROWS_PER_TILE)], out_hbm.at[c, s])


_spmm_sc = functools.partial(
    pl.kernel,
    out_type=jax.ShapeDtypeStruct((NC, NS, ROWS_PER_TILE, DIM), jnp.float32),
    mesh=plsc.VectorSubcoreMesh(core_axis_name="c", subcore_axis_name="s",
                                num_cores=NC),
    scratch_types=[
        pltpu.VMEM((CHUNKS_PER_TILE, CH), jnp.int32),
        pltpu.VMEM((CH,), jnp.int32),
        pltpu.VMEM((CH,), jnp.int32),
        pltpu.VMEM((CH,), jnp.float32),
        pltpu.VMEM((CH,), jnp.float32),
        pltpu.VMEM((CH, DIM), jnp.float32),
        pltpu.VMEM((CH, DIM), jnp.float32),
        pltpu.VMEM((ZROWS, DIM), jnp.float32),
        pltpu.VMEM_SHARED((N_NODES, DIM), jnp.float32),
        pltpu.SemaphoreType.DMA,
        pltpu.SemaphoreType.DMA,
        pltpu.SemaphoreType.DMA,
        pltpu.SemaphoreType.DMA,
        pltpu.SemaphoreType.DMA,
        pltpu.SemaphoreType.DMA,
    ],
)(_spmm_body)


def _linear_block(parts_ref, w_ref, o_ref, *, relu):
    p = parts_ref[0] + parts_ref[1]
    y = lax.dot_general(p, w_ref[...], (((1,), (1,)), ((), ())),
                        preferred_element_type=jnp.float32)
    if relu:
        y = jnp.maximum(y, 0.0)
    o_ref[...] = y


def _tc_linear(parts, w, relu):
    bn = 2000
    return pl.pallas_call(
        functools.partial(_linear_block, relu=relu),
        grid=(N_NODES // bn,),
        in_specs=[
            pl.BlockSpec((NC, bn, DIM), lambda i: (0, i, 0)),
            pl.BlockSpec((DIM, DIM), lambda i: (0, 0)),
        ],
        out_specs=pl.BlockSpec((bn, DIM), lambda i: (i, 0)),
        out_shape=jax.ShapeDtypeStruct((N_NODES, DIM), jnp.float32),
    )(parts, w)


@jax.jit
def kernel(x, edge_index, edge_weight, W1, W2):
    src = edge_index[0].astype(jnp.int32).reshape(NC * NS, CHUNKS_PER_TILE, CH)
    dst = edge_index[1].astype(jnp.int32).reshape(NC * NS, CHUNKS_PER_TILE, CH)
    ew = edge_weight.reshape(NC * NS, CHUNKS_PER_TILE, CH)

    p1 = _spmm_sc(x, src, dst, ew).reshape(NC, N_NODES, DIM)
    h = _tc_linear(p1, W1, relu=True)
    p2 = _spmm_sc(h, src, dst, ew).reshape(NC, N_NODES, DIM)
    return _tc_linear(p2, W2, relu=False)


# speedup vs baseline: 11.4708x; 1.1116x over previous
import functools

import jax
import jax.numpy as jnp
from jax import lax
from jax.experimental import pallas as pl
from jax.experimental.pallas import tpu as pltpu
import jax.experimental.pallas.tpu_sc as plsc

N_NODES = 10000
N_EDGES = 320000
DIM = 128

NC = 2
NS = 16
L = 16

CH = 80
CHUNKS_PER_TILE = N_EDGES // (NC * NS * CH)
ROWS_PER_TILE = N_NODES // NS
ZROWS = 5
NBUF = 3


def _spmm_body(x_hbm, src_hbm, dst_hbm, w_hbm, out_hbm,
               sb0, sb1, sb2, db0, db1, db2,
               wb0, wb1, wb2, rw0, rw1, rw2,
               zbuf, acc, ssrc, sg, sw, sd, ss):
    sb = [sb0, sb1, sb2]
    db = [db0, db1, db2]
    wb = [wb0, wb1, wb2]
    rows = [rw0, rw1, rw2]
    c = lax.axis_index("c")
    s = lax.axis_index("s")
    wid = c * NS + s

    def issue_src(q, b):
        pltpu.async_copy(src_hbm.at[wid, q], sb[b], ssrc[b])

    def wait_src(q, b):
        pltpu.make_async_copy(src_hbm.at[wid, q], sb[b], ssrc[b]).wait()

    def issue_dw(q, b):
        pltpu.async_copy(dst_hbm.at[wid, q], db[b], sd[b])
        pltpu.async_copy(w_hbm.at[wid, q], wb[b], sw[b])

    def wait_dw(q, b):
        pltpu.make_async_copy(dst_hbm.at[wid, q], db[b], sd[b]).wait()
        pltpu.make_async_copy(w_hbm.at[wid, q], wb[b], sw[b]).wait()

    def issue_gather(b):
        pltpu.async_copy(x_hbm.at[sb[b]], rows[b], sg[b])

    def wait_gather(b):
        pltpu.make_async_copy(x_hbm.at[sb[b]], rows[b], sg[b]).wait()

    def issue_scatter(b):
        pltpu.async_copy(rows[b], acc.at[db[b]], ss[b], add=True)

    def wait_scatter(b):
        pltpu.make_async_copy(rows[b], acc.at[db[b]], ss[b]).wait()

    def scale(b):
        rows_b = rows[b]
        wb_b = wb[b]

        def grp_body(eg, carry):
            wv = wb_b[pl.ds(eg * L, L)]
            for j in range(L):
                we = wv[j]
                e = eg * L + j
                for k in range(DIM // L):
                    sl = pl.ds(k * L, L)
                    rows_b[e, sl] = rows_b[e, sl] * we
            return carry

        lax.fori_loop(0, CH // L, grp_body, 0)

    for b in range(NBUF):
        pltpu.sync_copy(src_hbm.at[wid, b], sb[b])
    issue_dw(0, 0)
    issue_dw(1, 1)
    issue_gather(0)
    issue_gather(1)

    zero16 = jnp.zeros((L,), jnp.float32)
    for i in range(ZROWS):
        for k in range(DIM // L):
            zbuf[i, pl.ds(k * L, L)] = zero16
    base_r = s * ROWS_PER_TILE

    def zloop(j, carry):
        pltpu.sync_copy(zbuf, acc.at[pl.ds(base_r + j * ZROWS, ZROWS)])
        return carry

    lax.fori_loop(0, ROWS_PER_TILE // ZROWS, zloop, 0)
    plsc.subcore_barrier()

    wait_gather(0)
    wait_dw(0, 0)
    scale(0)
    issue_scatter(0)
    issue_dw(2, 2)
    issue_gather(2)
    issue_src(3, 0)
    wait_gather(1)
    wait_dw(1, 1)
    scale(1)
    wait_scatter(0)
    issue_scatter(1)
    wait_src(3, 0)
    issue_dw(3, 0)
    issue_gather(0)
    issue_src(4, 1)

    n_steps = (CHUNKS_PER_TILE - 2) // NBUF

    def pipe_body(t, carry):
        for j in range(NBUF):
            q = 2 + NBUF * t + j
            b = (2 + j) % NBUF
            bp = (j + 1) % NBUF
            wait_gather(b)
            wait_dw(q, b)
            scale(b)
            wait_scatter(bp)
            issue_scatter(b)
            if j == 0:
                wait_src(q + 2, bp)
                issue_dw(q + 2, bp)
                issue_gather(bp)
            else:
                @pl.when(t < n_steps - 1)
                def _():
                    wait_src(q + 2, bp)
                    issue_dw(q + 2, bp)
                    issue_gather(bp)

            @pl.when(t < n_steps - 1)
            def _():
                issue_src(q + 3, b)
        return carry

    lax.fori_loop(0, n_steps, pipe_body, 0)

    wait_scatter(1)

    plsc.subcore_barrier()

    pltpu.sync_copy(acc.at[pl.ds(base_r, ROWS_PER_TILE)], out_hbm.at[c, s])


_spmm_sc = functools.partial(
    pl.kernel,
    out_type=jax.ShapeDtypeStruct((NC, NS, ROWS_PER_TILE, DIM), jnp.float32),
    mesh=plsc.VectorSubcoreMesh(core_axis_name="c", subcore_axis_name="s",
                                num_cores=NC),
    scratch_types=[
        pltpu.VMEM((CH,), jnp.int32),
        pltpu.VMEM((CH,), jnp.int32),
        pltpu.VMEM((CH,), jnp.int32),
        pltpu.VMEM((CH,), jnp.int32),
        pltpu.VMEM((CH,), jnp.int32),
        pltpu.VMEM((CH,), jnp.int32),
        pltpu.VMEM((CH,), jnp.float32),
        pltpu.VMEM((CH,), jnp.float32),
        pltpu.VMEM((CH,), jnp.float32),
        pltpu.VMEM((CH, DIM), jnp.float32),
        pltpu.VMEM((CH, DIM), jnp.float32),
        pltpu.VMEM((CH, DIM), jnp.float32),
        pltpu.VMEM((ZROWS, DIM), jnp.float32),
        pltpu.VMEM_SHARED((N_NODES, DIM), jnp.float32),
        [pltpu.SemaphoreType.DMA] * NBUF,
        [pltpu.SemaphoreType.DMA] * NBUF,
        [pltpu.SemaphoreType.DMA] * NBUF,
        [pltpu.SemaphoreType.DMA] * NBUF,
        [pltpu.SemaphoreType.DMA] * NBUF,
    ],
)(_spmm_body)


def _linear_block(parts_ref, w_ref, o_ref, *, relu):
    p = parts_ref[0] + parts_ref[1]
    y = lax.dot_general(p, w_ref[...], (((1,), (1,)), ((), ())),
                        preferred_element_type=jnp.float32)
    if relu:
        y = jnp.maximum(y, 0.0)
    o_ref[...] = y


def _tc_linear(parts, w, relu):
    bn = 2000
    return pl.pallas_call(
        functools.partial(_linear_block, relu=relu),
        grid=(N_NODES // bn,),
        in_specs=[
            pl.BlockSpec((NC, bn, DIM), lambda i: (0, i, 0)),
            pl.BlockSpec((DIM, DIM), lambda i: (0, 0)),
        ],
        out_specs=pl.BlockSpec((bn, DIM), lambda i: (i, 0)),
        out_shape=jax.ShapeDtypeStruct((N_NODES, DIM), jnp.float32),
    )(parts, w)


@jax.jit
def kernel(x, edge_index, edge_weight, W1, W2):
    src = edge_index[0].astype(jnp.int32).reshape(NC * NS, CHUNKS_PER_TILE, CH)
    dst = edge_index[1].astype(jnp.int32).reshape(NC * NS, CHUNKS_PER_TILE, CH)
    ew = edge_weight.reshape(NC * NS, CHUNKS_PER_TILE, CH)

    p1 = _spmm_sc(x, src, dst, ew).reshape(NC, N_NODES, DIM)
    h = _tc_linear(p1, W1, relu=True)
    p2 = _spmm_sc(h, src, dst, ew).reshape(NC, N_NODES, DIM)
    return _tc_linear(p2, W2, relu=False)
